# speedup vs baseline: 1.0154x; 1.0154x over previous
import functools

import jax
import jax.numpy as jnp
from jax import lax
from jax.experimental import pallas as pl
from jax.experimental.pallas import tpu as pltpu
from jax.experimental.pallas import tpu_sc as plsc

NC = 2
NS = 16
L = 16
NW = NC * NS
BATCH = 4096
DIM = 128
BPW = BATCH // NW

_MESH = plsc.VectorSubcoreMesh(core_axis_name="c", subcore_axis_name="s")


@functools.partial(
    pl.kernel,
    out_type=jax.ShapeDtypeStruct((NW, L), jnp.float32),
    mesh=_MESH,
    scratch_types=[
        pltpu.VMEM((BPW,), jnp.int32),
        pltpu.VMEM((BPW, DIM), jnp.float32),
        pltpu.VMEM((BPW, DIM), jnp.float32),
        pltpu.VMEM((L,), jnp.float32),
    ],
)
def _center_loss_partials(inp_hbm, tgt_hbm, cen_hbm, out_hbm,
                          idx_v, rows_v, in_v, acc_v):
    wid = lax.axis_index("s") * NC + lax.axis_index("c")
    base = wid * BPW
    pltpu.sync_copy(tgt_hbm.at[pl.ds(base, BPW)], idx_v)
    pltpu.sync_copy(cen_hbm.at[idx_v], rows_v)
    pltpu.sync_copy(inp_hbm.at[pl.ds(base, BPW), :], in_v)

    def row_body(i, acc):
        for j in range(DIM // L):
            a = in_v[i, pl.ds(j * L, L)]
            b = rows_v[i, pl.ds(j * L, L)]
            d = a - b
            acc = acc + d * d
        return acc

    acc = lax.fori_loop(0, BPW, row_body, jnp.zeros((L,), jnp.float32))
    acc_v[...] = acc
    pltpu.sync_copy(acc_v, out_hbm.at[wid])


@jax.jit
def kernel(input, target, centers):
    partials = _center_loss_partials(input, target.astype(jnp.int32), centers)
    return jnp.sum(partials) / (2.0 * BATCH)


# speedup vs baseline: 1.0368x; 1.0211x over previous
import functools

import jax
import jax.numpy as jnp
from jax import lax
from jax.experimental import pallas as pl
from jax.experimental.pallas import tpu as pltpu
from jax.experimental.pallas import tpu_sc as plsc

NC = 2
NS = 16
L = 16
NW = NC * NS
BATCH = 4096
DIM = 128
BPW = BATCH // NW
CH = 32
NCH = BPW // CH

_MESH = plsc.VectorSubcoreMesh(core_axis_name="c", subcore_axis_name="s")


@functools.partial(
    pl.kernel,
    out_type=jax.ShapeDtypeStruct((NW, L), jnp.float32),
    mesh=_MESH,
    scratch_types=[
        pltpu.VMEM((BPW,), jnp.int32),
        pltpu.VMEM((2, CH, DIM), jnp.float32),
        pltpu.VMEM((BPW, DIM), jnp.float32),
        pltpu.VMEM((L,), jnp.float32),
        pltpu.SemaphoreType.DMA,
        pltpu.SemaphoreType.DMA,
        pltpu.SemaphoreType.DMA,
    ],
)
def _center_loss_partials(inp_hbm, tgt_hbm, cen_hbm, out_hbm,
                          idx_v, rows_v, in_v, acc_v,
                          sem_in, sem_g0, sem_g1):
    wid = lax.axis_index("s") * NC + lax.axis_index("c")
    base = wid * BPW

    in_dma = pltpu.async_copy(inp_hbm.at[pl.ds(base, BPW), :], in_v, sem_in)
    pltpu.sync_copy(tgt_hbm.at[pl.ds(base, BPW)], idx_v)

    gsems = (sem_g0, sem_g1)
    gathers = [None, None]
    gathers[0] = pltpu.async_copy(
        cen_hbm.at[idx_v.at[pl.ds(0, CH)]], rows_v.at[0], gsems[0])
    gathers[1] = pltpu.async_copy(
        cen_hbm.at[idx_v.at[pl.ds(CH, CH)]], rows_v.at[1], gsems[1])
    in_dma.wait()

    def chunk_rows(k, buf, acc):
        def row_body(i, acc):
            for j in range(DIM // L):
                a = in_v[k * CH + i, pl.ds(j * L, L)]
                b = rows_v[buf, i, pl.ds(j * L, L)]
                d = a - b
                acc = acc + d * d
            return acc
        return lax.fori_loop(0, CH, row_body, acc)

    acc = jnp.zeros((L,), jnp.float32)
    for k in range(NCH):
        buf = k % 2
        gathers[buf].wait()
        acc = chunk_rows(k, buf, acc)
        if k + 2 < NCH:
            gathers[buf] = pltpu.async_copy(
                cen_hbm.at[idx_v.at[pl.ds((k + 2) * CH, CH)]],
                rows_v.at[buf], gsems[buf])

    acc_v[...] = acc
    pltpu.sync_copy(acc_v, out_hbm.at[wid])


@jax.jit
def kernel(input, target, centers):
    partials = _center_loss_partials(input, target.astype(jnp.int32), centers)
    return jnp.sum(partials) / (2.0 * BATCH)


# speedup vs baseline: 1.0378x; 1.0010x over previous
import functools

import jax
import jax.numpy as jnp
from jax import lax
from jax.experimental import pallas as pl
from jax.experimental.pallas import tpu as pltpu
from jax.experimental.pallas import tpu_sc as plsc

NC = 2
NS = 16
L = 16
NW = NC * NS
BATCH = 4096
DIM = 128
BPW = BATCH // NW
CH = 32
NCH = BPW // CH

_MESH = plsc.VectorSubcoreMesh(core_axis_name="c", subcore_axis_name="s")


@functools.partial(
    pl.kernel,
    out_type=jax.ShapeDtypeStruct((NW, L), jnp.float32),
    mesh=_MESH,
    scratch_types=[
        pltpu.VMEM((BPW,), jnp.int32),
        pltpu.VMEM((2, CH, DIM), jnp.float32),
        pltpu.VMEM((BPW, DIM), jnp.float32),
        pltpu.VMEM((L,), jnp.float32),
        pltpu.SemaphoreType.DMA,
        pltpu.SemaphoreType.DMA,
        pltpu.SemaphoreType.DMA,
    ],
)
def _center_loss_partials(inp_hbm, tgt_hbm, cen_hbm, out_hbm,
                          idx_v, rows_v, in_v, acc_v,
                          sem_in, sem_g0, sem_g1):
    wid = lax.axis_index("s") * NC + lax.axis_index("c")
    base = wid * BPW

    in_dma = pltpu.async_copy(inp_hbm.at[pl.ds(base, BPW), :], in_v, sem_in)
    pltpu.sync_copy(tgt_hbm.at[pl.ds(base, BPW)], idx_v)

    gsems = (sem_g0, sem_g1)
    gathers = [None, None]
    gathers[0] = pltpu.async_copy(
        cen_hbm.at[idx_v.at[pl.ds(0, CH)]], rows_v.at[0], gsems[0])
    gathers[1] = pltpu.async_copy(
        cen_hbm.at[idx_v.at[pl.ds(CH, CH)]], rows_v.at[1], gsems[1])
    in_dma.wait()

    def chunk_rows(k, buf, accs):
        def row_body(i, accs):
            new = []
            for j in range(DIM // L):
                a = in_v[k * CH + i, pl.ds(j * L, L)]
                b = rows_v[buf, i, pl.ds(j * L, L)]
                d = a - b
                new.append(accs[j] + d * d)
            return tuple(new)
        return lax.fori_loop(0, CH, row_body, accs)

    accs = tuple(jnp.zeros((L,), jnp.float32) for _ in range(DIM // L))
    for k in range(NCH):
        buf = k % 2
        gathers[buf].wait()
        accs = chunk_rows(k, buf, accs)
        if k + 2 < NCH:
            gathers[buf] = pltpu.async_copy(
                cen_hbm.at[idx_v.at[pl.ds((k + 2) * CH, CH)]],
                rows_v.at[buf], gsems[buf])

    a01, a23 = accs[0] + accs[1], accs[2] + accs[3]
    a45, a67 = accs[4] + accs[5], accs[6] + accs[7]
    acc_v[...] = (a01 + a23) + (a45 + a67)
    pltpu.sync_copy(acc_v, out_hbm.at[wid])


@jax.jit
def kernel(input, target, centers):
    partials = _center_loss_partials(input, target.astype(jnp.int32), centers)
    return jnp.sum(partials) / (2.0 * BATCH)


# speedup vs baseline: 1.0428x; 1.0049x over previous
import functools

import jax
import jax.numpy as jnp
from jax import lax
from jax.experimental import pallas as pl
from jax.experimental.pallas import tpu as pltpu
from jax.experimental.pallas import tpu_sc as plsc

NC = 2
NS = 16
L = 16
NW = NC * NS
BATCH = 4096
DIM = 128
BPW = BATCH // NW
CH = 32
NCH = BPW // CH

_MESH = plsc.VectorSubcoreMesh(core_axis_name="c", subcore_axis_name="s")


@functools.partial(
    pl.kernel,
    out_type=jax.ShapeDtypeStruct((NW, L), jnp.float32),
    mesh=_MESH,
    scratch_types=[
        pltpu.VMEM((BPW,), jnp.int32),
        pltpu.VMEM((BPW, DIM), jnp.float32),
        pltpu.VMEM((BPW, DIM), jnp.float32),
        pltpu.VMEM((L,), jnp.float32),
    ]
    + [pltpu.SemaphoreType.DMA] * (2 * NCH),
)
def _center_loss_partials(inp_hbm, tgt_hbm, cen_hbm, out_hbm,
                          idx_v, rows_v, in_v, acc_v, *sems):
    in_sems, g_sems = sems[:NCH], sems[NCH:]
    wid = lax.axis_index("s") * NC + lax.axis_index("c")
    base = wid * BPW

    pltpu.sync_copy(tgt_hbm.at[pl.ds(base, BPW)], idx_v)
    copies = []
    for k in range(NCH):
        r = pl.ds(k * CH, CH)
        g = pltpu.async_copy(
            cen_hbm.at[idx_v.at[r]], rows_v.at[r], g_sems[k])
        i = pltpu.async_copy(
            inp_hbm.at[pl.ds(base + k * CH, CH), :], in_v.at[r], in_sems[k])
        copies.append((g, i))

    def chunk_rows(k, accs):
        def row_body(i, accs):
            new = []
            for j in range(DIM // L):
                a = in_v[k * CH + i, pl.ds(j * L, L)]
                b = rows_v[k * CH + i, pl.ds(j * L, L)]
                d = a - b
                new.append(accs[j] + d * d)
            return tuple(new)
        return lax.fori_loop(0, CH, row_body, accs)

    accs = tuple(jnp.zeros((L,), jnp.float32) for _ in range(DIM // L))
    for k in range(NCH):
        copies[k][0].wait()
        copies[k][1].wait()
        accs = chunk_rows(k, accs)

    a01, a23 = accs[0] + accs[1], accs[2] + accs[3]
    a45, a67 = accs[4] + accs[5], accs[6] + accs[7]
    acc_v[...] = (a01 + a23) + (a45 + a67)
    pltpu.sync_copy(acc_v, out_hbm.at[wid])


@jax.jit
def kernel(input, target, centers):
    partials = _center_loss_partials(input, target.astype(jnp.int32), centers)
    return jnp.sum(partials) / (2.0 * BATCH)
